# Initial kernel scaffold; baseline (speedup 1.0000x reference)
import jax
import jax.numpy as jnp
from jax.experimental import pallas as pl


def kernel(x, edge_index, batch, W1, b1, W2, b2, W3, b3, W4, b4, Wl1, bl1, Wl2, bl2):
    raise NotImplementedError("write your pallas kernel here")


# speedup vs baseline: 1.0000x; 1.0000x over previous
import jax, jax.numpy as jnp
from jax.experimental import pallas as pl

N = 100000
E = 1600000
G = 128


def _gcn_conv(x, src, dst, W, b, n):
    deg = jax.ops.segment_sum(jnp.ones_like(src, dtype=x.dtype), dst, num_segments=n)
    dinv = jax.lax.rsqrt(jnp.maximum(deg, 1.0))
    norm = dinv[src] * dinv[dst]
    h = x @ W
    msg = h[src] * norm[:, None]
    return jax.ops.segment_sum(msg, dst, num_segments=n) + b


def _head_kernel(pooled_ref, wl1_ref, bl1_ref, wl2_ref, bl2_ref, out_ref):
    z = jnp.maximum(pooled_ref[...] @ wl1_ref[...] + bl1_ref[...], 0.0)
    out_ref[...] = z @ wl2_ref[...] + bl2_ref[...]


def kernel(x, edge_index, batch, W1, b1, W2, b2, W3, b3, W4, b4, Wl1, bl1, Wl2, bl2):
    loop = jnp.arange(N, dtype=edge_index.dtype)
    src = jnp.concatenate([edge_index[0], loop])
    dst = jnp.concatenate([edge_index[1], loop])
    h = jax.nn.relu(_gcn_conv(x, src, dst, W1, b1, N))
    h = jax.nn.relu(_gcn_conv(h, src, dst, W2, b2, N))
    h = jax.nn.relu(_gcn_conv(h, src, dst, W3, b3, N))
    h = _gcn_conv(h, src, dst, W4, b4, N)
    sums = jax.ops.segment_sum(h, batch, num_segments=G)
    cnt = jax.ops.segment_sum(jnp.ones((N,), h.dtype), batch, num_segments=G)
    pooled = sums / jnp.maximum(cnt, 1.0)[:, None]
    out = pl.pallas_call(
        _head_kernel,
        out_shape=jax.ShapeDtypeStruct((G, 9), jnp.float32),
    )(pooled, Wl1, bl1, Wl2, bl2)
    return out


# speedup vs baseline: 2.7935x; 2.7934x over previous
import functools
import jax
import jax.numpy as jnp
from jax import lax
from jax.experimental import pallas as pl
from jax.experimental.pallas import tpu as pltpu
from jax.experimental.pallas import tpu_sc as plsc

N = 100000
E = 1600000
G = 128

NW = 32
EB = 128
NB = 392
EW = NB * EB
EPAD = NW * EW
NPAD = 100096
RPT = NPAD // 16
BN = 2000
NG = N // BN


def _make_edge_agg(C):
    mesh = plsc.VectorSubcoreMesh(core_axis_name="c", subcore_axis_name="s")

    @functools.partial(
        pl.kernel,
        mesh=mesh,
        compiler_params=pltpu.CompilerParams(use_tc_tiling_on_sc=False),
        out_type=jax.ShapeDtypeStruct((C, 2, NPAD, 16), jnp.float32),
        scratch_types=[
            pltpu.VMEM((EB,), jnp.int32),
            pltpu.VMEM((1, EB), jnp.int32),
            pltpu.VMEM((EB, 16), jnp.float32),
            pltpu.VMEM_SHARED((NPAD, 16), jnp.float32),
            pltpu.SemaphoreType.DMA,
        ],
    )
    def agg(u_hbm, srcb_hbm, dstb_hbm, zeros_hbm, out_hbm,
            src_v, dst_v, rows_v, acc_s, sem):
        core = lax.axis_index("c")
        sub = lax.axis_index("s")
        wid = sub * 2 + core
        for c in range(C):
            pltpu.sync_copy(zeros_hbm.at[pl.ds(sub * RPT, RPT)],
                            acc_s.at[pl.ds(sub * RPT, RPT)])
            plsc.subcore_barrier()

            def body(j, carry):
                off = pl.multiple_of(wid * EW + j * EB, 8)
                pltpu.sync_copy(srcb_hbm.at[pl.ds(off, EB)], src_v)
                pltpu.sync_copy(dstb_hbm.at[pl.ds(wid * NB + j, 1)], dst_v)
                pltpu.async_copy(u_hbm.at[c].at[src_v], rows_v, sem).wait()
                pltpu.sync_copy(rows_v, acc_s.at[dst_v.at[0]], add=True)
                return carry

            lax.fori_loop(0, NB, body, 0)
            plsc.subcore_barrier()
            pltpu.sync_copy(acc_s.at[pl.ds(sub * RPT, RPT)],
                            out_hbm.at[c, core, pl.ds(sub * RPT, RPT)])

    return agg


def _node_spec(w):
    return pl.BlockSpec((BN, w), lambda i: (i, 0))


def _full_spec(shape):
    return pl.BlockSpec(shape, lambda i: tuple(0 for _ in shape))


def _dense_first_kernel(p0, p1, u, dinv, wa, ba, wb, o):
    t = dinv[...] * (p0[...] + p1[...] + u[...])
    h = jnp.dot(t, wa[...], preferred_element_type=jnp.float32) + ba[...]
    h = jnp.maximum(h, 0.0)
    o[...] = dinv[...] * jnp.dot(h, wb[...], preferred_element_type=jnp.float32)


def _dense_mid_kernel(p0, p1, u, dinv, b, wb, o):
    h = jnp.maximum(dinv[...] * (p0[...] + p1[...] + u[...]) + b[...], 0.0)
    o[...] = dinv[...] * jnp.dot(h, wb[...], preferred_element_type=jnp.float32)


def _pool_head_kernel(p0, p1, u, dinv, b4, bidx, wl1, bl1, wl2, bl2, o, acc):
    i = pl.program_id(0)
    g4 = dinv[...] * (p0[...] + p1[...] + u[...]) + b4[...]
    oh = (bidx[...] == lax.broadcasted_iota(jnp.int32, (BN, G), 1))
    oh = oh.astype(jnp.float32)
    z = jnp.concatenate([g4, jnp.ones((BN, 1), jnp.float32)], axis=1)
    part = lax.dot_general(oh, z, (((0,), (0,)), ((), ())),
                           preferred_element_type=jnp.float32)

    @pl.when(i == 0)
    def _():
        acc[...] = jnp.zeros_like(acc)

    acc[...] += part

    @pl.when(i == pl.num_programs(0) - 1)
    def _():
        pooled = acc[:, :64] / jnp.maximum(acc[:, 64:65], 1.0)
        zz = jnp.maximum(
            jnp.dot(pooled, wl1[...], preferred_element_type=jnp.float32)
            + bl1[...], 0.0)
        o[...] = jnp.dot(zz, wl2[...],
                         preferred_element_type=jnp.float32) + bl2[...]


def _chunked(u, c):
    return u.reshape(N, c, 16).transpose(1, 0, 2)


def _unchunk(q):
    c = q.shape[0]
    return q.transpose(1, 0, 2).reshape(N, c * 16)


def kernel(x, edge_index, batch, W1, b1, W2, b2, W3, b3, W4, b4,
           Wl1, bl1, Wl2, bl2):
    f32 = jnp.float32
    pad = EPAD - E
    srcb = jnp.concatenate([edge_index[0], jnp.zeros((pad,), jnp.int32)])
    dstb = jnp.concatenate(
        [edge_index[1], jnp.full((pad,), N, jnp.int32)]).reshape(NW * NB, EB)
    zeros16 = jnp.zeros((NPAD, 16), f32)

    agg1 = _make_edge_agg(1)
    agg4 = _make_edge_agg(4)
    agg8 = _make_edge_agg(8)

    pdeg = agg1(jnp.ones((1, N, 16), f32), srcb, dstb, zeros16)
    deg = pdeg[0, 0, :N, 0] + pdeg[0, 1, :N, 0] + 1.0
    dinv = lax.rsqrt(deg)
    dinv2 = dinv[:, None]

    u1 = jnp.pad(dinv2 * x, ((0, 0), (0, 13)))
    q1 = agg1(u1[None], srcb, dstb, zeros16)
    W1p = jnp.zeros((16, 128), f32).at[:3].set(W1)

    u2 = pl.pallas_call(
        _dense_first_kernel,
        grid=(NG,),
        in_specs=[_node_spec(16), _node_spec(16), _node_spec(16),
                  _node_spec(1), _full_spec((16, 128)), _full_spec((1, 128)),
                  _full_spec((128, 128))],
        out_specs=_node_spec(128),
        out_shape=jax.ShapeDtypeStruct((N, 128), f32),
    )(q1[0, 0, :N], q1[0, 1, :N], u1, dinv2, W1p, b1[None], W2)

    def mid(q, u, b, wb, wout):
        return pl.pallas_call(
            _dense_mid_kernel,
            grid=(NG,),
            in_specs=[_node_spec(128), _node_spec(128), _node_spec(128),
                      _node_spec(1), _full_spec((1, 128)),
                      _full_spec((128, wout))],
            out_specs=_node_spec(wout),
            out_shape=jax.ShapeDtypeStruct((N, wout), f32),
        )(_unchunk(q[:, 0, :N]), _unchunk(q[:, 1, :N]), u, dinv2, b[None], wb)

    q2 = agg8(_chunked(u2, 8), srcb, dstb, zeros16)
    u3 = mid(q2, u2, b2, W3, 128)

    q3 = agg8(_chunked(u3, 8), srcb, dstb, zeros16)
    u4 = mid(q3, u3, b3, W4, 64)

    q4 = agg4(_chunked(u4, 4), srcb, dstb, zeros16)

    out = pl.pallas_call(
        _pool_head_kernel,
        grid=(NG,),
        in_specs=[_node_spec(64), _node_spec(64), _node_spec(64),
                  _node_spec(1), _full_spec((1, 64)), _node_spec(1),
                  _full_spec((64, 32)), _full_spec((1, 32)),
                  _full_spec((32, 9)), _full_spec((1, 9))],
        out_specs=_full_spec((G, 9)),
        out_shape=jax.ShapeDtypeStruct((G, 9), f32),
        scratch_shapes=[pltpu.VMEM((G, 65), f32)],
    )(_unchunk(q4[:, 0, :N]), _unchunk(q4[:, 1, :N]), u4, dinv2, b4[None],
      batch[:, None], Wl1, bl1[None], Wl2, bl2[None])
    return out


# speedup vs baseline: 3.9597x; 1.4175x over previous
import functools
import jax
import jax.numpy as jnp
from jax import lax
from jax.experimental import pallas as pl
from jax.experimental.pallas import tpu as pltpu
from jax.experimental.pallas import tpu_sc as plsc

N = 100000
E = 1600000
G = 128

NW = 32
EB = 128
NB = 392
EW = NB * EB
EPAD = NW * EW
NPAD = 100096
RPT = NPAD // 16
BN = 2000
NG = N // BN


def _make_edge_agg(C):
    mesh = plsc.VectorSubcoreMesh(core_axis_name="c", subcore_axis_name="s")

    @functools.partial(
        pl.kernel,
        mesh=mesh,
        compiler_params=pltpu.CompilerParams(use_tc_tiling_on_sc=False),
        out_type=jax.ShapeDtypeStruct((C, 2, NPAD, 16), jnp.float32),
        scratch_types=[
            pltpu.VMEM((2, EB), jnp.int32),
            pltpu.VMEM((2, EB), jnp.int32),
            pltpu.VMEM((2, EB, 16), jnp.float32),
            pltpu.VMEM_SHARED((NPAD, 16), jnp.float32),
            pltpu.SemaphoreType.DMA,
            pltpu.SemaphoreType.DMA,
        ],
    )
    def agg(u_hbm, srcb_hbm, dstb_hbm, zeros_hbm, out_hbm,
            src_v, dst_v, rows_v, acc_s, sem0, sem1):
        core = lax.axis_index("c")
        sub = lax.axis_index("s")
        wid = sub * 2 + core
        sems = (sem0, sem1)

        for c in range(C):
            pltpu.sync_copy(zeros_hbm.at[pl.ds(sub * RPT, RPT)],
                            acc_s.at[pl.ds(sub * RPT, RPT)])
            plsc.subcore_barrier()

            def fetch(j, sl):
                off = pl.multiple_of(wid * EW + j * EB, 8)
                pltpu.sync_copy(srcb_hbm.at[pl.ds(off, EB)], src_v.at[sl])
                pltpu.sync_copy(dstb_hbm.at[pl.ds(off, EB)], dst_v.at[sl])
                pltpu.async_copy(u_hbm.at[c].at[src_v.at[sl]],
                                 rows_v.at[sl], sems[sl])

            def step(j, sl):
                nxt = 1 - sl

                @pl.when(j + 1 < NB)
                def _():
                    fetch(j + 1, nxt)

                pltpu.make_async_copy(u_hbm.at[c].at[src_v.at[sl]],
                                      rows_v.at[sl], sems[sl]).wait()
                pltpu.sync_copy(rows_v.at[sl], acc_s.at[dst_v.at[sl]],
                                add=True)

            fetch(0, 0)

            def body(g, carry):
                step(2 * g, 0)
                step(2 * g + 1, 1)
                return carry

            lax.fori_loop(0, NB // 2, body, 0)
            plsc.subcore_barrier()
            pltpu.sync_copy(acc_s.at[pl.ds(sub * RPT, RPT)],
                            out_hbm.at[c, core, pl.ds(sub * RPT, RPT)])

    return agg


def _node_spec(w):
    return pl.BlockSpec((BN, w), lambda i: (i, 0))


def _full_spec(shape):
    return pl.BlockSpec(shape, lambda i: tuple(0 for _ in shape))


def _dense_first_kernel(p0, p1, u, dinv, wa, ba, wb, o):
    t = dinv[...] * (p0[...] + p1[...] + u[...])
    h = jnp.dot(t, wa[...], preferred_element_type=jnp.float32) + ba[...]
    h = jnp.maximum(h, 0.0)
    o[...] = dinv[...] * jnp.dot(h, wb[...], preferred_element_type=jnp.float32)


def _dense_mid_kernel(p0, p1, u, dinv, b, wb, o):
    h = jnp.maximum(dinv[...] * (p0[...] + p1[...] + u[...]) + b[...], 0.0)
    o[...] = dinv[...] * jnp.dot(h, wb[...], preferred_element_type=jnp.float32)


def _pool_head_kernel(p0, p1, u, dinv, b4, bidx, wl1, bl1, wl2, bl2, o, acc):
    i = pl.program_id(0)
    g4 = dinv[...] * (p0[...] + p1[...] + u[...]) + b4[...]
    oh = (bidx[...] == lax.broadcasted_iota(jnp.int32, (BN, G), 1))
    oh = oh.astype(jnp.float32)
    z = jnp.concatenate([g4, jnp.ones((BN, 1), jnp.float32)], axis=1)
    part = lax.dot_general(oh, z, (((0,), (0,)), ((), ())),
                           preferred_element_type=jnp.float32)

    @pl.when(i == 0)
    def _():
        acc[...] = jnp.zeros_like(acc)

    acc[...] += part

    @pl.when(i == pl.num_programs(0) - 1)
    def _():
        pooled = acc[:, :64] / jnp.maximum(acc[:, 64:65], 1.0)
        zz = jnp.maximum(
            jnp.dot(pooled, wl1[...], preferred_element_type=jnp.float32)
            + bl1[...], 0.0)
        o[...] = jnp.dot(zz, wl2[...],
                         preferred_element_type=jnp.float32) + bl2[...]


def _chunked(u, c):
    return u.reshape(N, c, 16).transpose(1, 0, 2)


def _unchunk(q):
    c = q.shape[0]
    return q.transpose(1, 0, 2).reshape(N, c * 16)


def kernel(x, edge_index, batch, W1, b1, W2, b2, W3, b3, W4, b4,
           Wl1, bl1, Wl2, bl2):
    f32 = jnp.float32
    pad = EPAD - E
    srcb = jnp.concatenate([edge_index[0], jnp.zeros((pad,), jnp.int32)])
    dstb = jnp.concatenate([edge_index[1], jnp.full((pad,), N, jnp.int32)])
    zeros16 = jnp.zeros((NPAD, 16), f32)

    agg1 = _make_edge_agg(1)
    agg4 = _make_edge_agg(4)
    agg8 = _make_edge_agg(8)

    pdeg = agg1(jnp.ones((1, N, 16), f32), srcb, dstb, zeros16)
    deg = pdeg[0, 0, :N, 0] + pdeg[0, 1, :N, 0] + 1.0
    dinv = lax.rsqrt(deg)
    dinv2 = dinv[:, None]

    u1 = jnp.pad(dinv2 * x, ((0, 0), (0, 13)))
    q1 = agg1(u1[None], srcb, dstb, zeros16)
    W1p = jnp.zeros((16, 128), f32).at[:3].set(W1)

    u2 = pl.pallas_call(
        _dense_first_kernel,
        grid=(NG,),
        in_specs=[_node_spec(16), _node_spec(16), _node_spec(16),
                  _node_spec(1), _full_spec((16, 128)), _full_spec((1, 128)),
                  _full_spec((128, 128))],
        out_specs=_node_spec(128),
        out_shape=jax.ShapeDtypeStruct((N, 128), f32),
    )(q1[0, 0, :N], q1[0, 1, :N], u1, dinv2, W1p, b1[None], W2)

    def mid(q, u, b, wb, wout):
        return pl.pallas_call(
            _dense_mid_kernel,
            grid=(NG,),
            in_specs=[_node_spec(128), _node_spec(128), _node_spec(128),
                      _node_spec(1), _full_spec((1, 128)),
                      _full_spec((128, wout))],
            out_specs=_node_spec(wout),
            out_shape=jax.ShapeDtypeStruct((N, wout), f32),
        )(_unchunk(q[:, 0, :N]), _unchunk(q[:, 1, :N]), u, dinv2, b[None], wb)

    q2 = agg8(_chunked(u2, 8), srcb, dstb, zeros16)
    u3 = mid(q2, u2, b2, W3, 128)

    q3 = agg8(_chunked(u3, 8), srcb, dstb, zeros16)
    u4 = mid(q3, u3, b3, W4, 64)

    q4 = agg4(_chunked(u4, 4), srcb, dstb, zeros16)

    out = pl.pallas_call(
        _pool_head_kernel,
        grid=(NG,),
        in_specs=[_node_spec(64), _node_spec(64), _node_spec(64),
                  _node_spec(1), _full_spec((1, 64)), _node_spec(1),
                  _full_spec((64, 32)), _full_spec((1, 32)),
                  _full_spec((32, 9)), _full_spec((1, 9))],
        out_specs=_full_spec((G, 9)),
        out_shape=jax.ShapeDtypeStruct((G, 9), f32),
        scratch_shapes=[pltpu.VMEM((G, 65), f32)],
    )(_unchunk(q4[:, 0, :N]), _unchunk(q4[:, 1, :N]), u4, dinv2, b4[None],
      batch[:, None], Wl1, bl1[None], Wl2, bl2[None])
    return out
